# baseline (speedup 1.0000x reference)
import functools

import jax
import jax.numpy as jnp
from jax import lax
from jax.experimental import pallas as pl
from jax.experimental.pallas import tpu as pltpu
from jax.experimental.pallas import tpu_sc as plsc

D_MODEL = 768
N_HEADS = 12
MAX_STATES = 64
ENERGY_BUDGET = 100.0
EPS = 1e-6


def _tc1_body(x_ref, wq_ref, wk_ref, wv_ref, we_ref, be_ref,
              q_ref, kvw_ref, idx_ref, cnt_ref, cn_ref, *, base):
    b = pl.program_id(0)
    s = pl.program_id(1)
    K = MAX_STATES
    D = D_MODEL

    x = x_ref[0]
    xh = x.astype(jnp.bfloat16)
    q = jnp.dot(xh, wq_ref[...], preferred_element_type=jnp.float32)
    k = jnp.dot(x, wk_ref[...], preferred_element_type=jnp.float32)
    v = jnp.dot(xh, wv_ref[...], preferred_element_type=jnp.float32)
    q_ref[0] = q.astype(jnp.bfloat16)

    dot = jnp.sum(x * we_ref[...], axis=-1, keepdims=True) + be_ref[0, 0]
    w = base * 2.0 * jax.nn.sigmoid(dot)

    kn = k / (jnp.sqrt(jnp.sum(k * k, axis=-1, keepdims=True)) + EPS)

    @pl.when(s == 0)
    def _():
        cn_ref[...] = kn[:K]
        cnt_ref[...] = jnp.zeros_like(cnt_ref)

    sims = lax.dot_general(kn, cn_ref[...], (((1,), (1,)), ((), ())),
                           preferred_element_type=jnp.float32)
    m = jnp.max(sims, axis=-1, keepdims=True)
    j = lax.broadcasted_iota(jnp.int32, sims.shape, 1)
    assign = jnp.min(jnp.where(sims >= m, j, K), axis=-1, keepdims=True)
    p = (j == assign).astype(jnp.float32)

    kvw_ref[:, :D] = k * w
    kvw_ref[:, D:] = v * w
    jrow = lax.broadcasted_iota(jnp.int32, (1, K), 1).astype(jnp.float32)
    arow = lax.dot_general(jrow, p, (((1,), (1,)), ((), ())),
                           preferred_element_type=jnp.float32)
    idx_ref[0] = arow.astype(jnp.int32) + b * K
    cnt_ref[0:1, :] += jnp.sum(p * w, axis=0, keepdims=True)


def _make_sc_scatter(BT, D2, NB):
    DC = 128
    NG = D2 // DC
    CH = 256
    half = BT // 2
    mesh = plsc.VectorSubcoreMesh(core_axis_name="c", subcore_axis_name="s")

    @functools.partial(
        pl.kernel, mesh=mesh,
        out_type=jax.ShapeDtypeStruct((2, NB, D2), jnp.float32),
        scratch_types=[
            pltpu.VMEM((CH,), jnp.int32),
            pltpu.VMEM((CH, DC), jnp.float32),
            pltpu.VMEM((NB, DC), jnp.float32),
        ],
    )
    def sc(kv_hbm, idx_hbm, out_hbm, idxbuf, rowbuf, acc):
        c = lax.axis_index("c")
        s = lax.axis_index("s")
        wid = s * 2 + c

        @pl.when(wid < 2 * NG)
        def _active():
            tr = wid // NG
            cg = wid - tr * NG
            col0 = pl.multiple_of(cg * DC, DC)
            tok0 = pl.multiple_of(tr * half, half)

            def zb(i, carry):
                for cb in range(DC // 16):
                    acc[i, pl.ds(cb * 16, 16)] = jnp.zeros((16,), jnp.float32)
                return carry
            lax.fori_loop(0, NB, zb, 0)

            def chunk(g, carry):
                base = pl.multiple_of(tok0 + g * CH, CH)
                pltpu.sync_copy(idx_hbm.at[pl.ds(base, CH)], idxbuf)
                pltpu.sync_copy(
                    kv_hbm.at[pl.ds(base, CH), pl.ds(col0, DC)], rowbuf)

                def grp(tt, carry2):
                    iv16 = idxbuf[pl.ds(tt * 16, 16)]
                    for lane in range(16):
                        r = iv16[lane]
                        trow = tt * 16 + lane
                        for cb in range(DC // 16):
                            sl = pl.ds(cb * 16, 16)
                            acc[r, sl] = acc[r, sl] + rowbuf[trow, sl]
                    return carry2
                lax.fori_loop(0, CH // 16, grp, 0)
                return carry
            lax.fori_loop(0, half // CH, chunk, 0)

            pltpu.sync_copy(acc, out_hbm.at[tr, :, pl.ds(col0, DC)])

    return sc


def _tc2_body(q_ref, skv_ref, cnt_ref, wo_ref, out_ref, acc_ref):
    H = N_HEADS
    D = D_MODEL
    dh = D // H
    inv = 1.0 / (cnt_ref[0, 0:1, :] + EPS)
    scale = inv / jnp.sqrt(jnp.float32(dh))
    skv = skv_ref[0, 0] + skv_ref[1, 0]
    q = q_ref[0]
    for h in range(H):
        lo = h * dh
        qh = q[:, lo:lo + dh]
        skh = skv[:, lo:lo + dh].astype(jnp.bfloat16)
        svh = skv[:, D + lo:D + lo + dh].astype(jnp.bfloat16)
        scores = lax.dot_general(qh, skh, (((1,), (1,)), ((), ())),
                                 preferred_element_type=jnp.float32) * scale
        mx = jnp.max(scores, axis=-1, keepdims=True)
        e = jnp.exp(scores - mx)
        attn = (e / jnp.sum(e, axis=-1, keepdims=True)) * inv
        acc_ref[:, lo:lo + dh] = jnp.dot(attn.astype(jnp.bfloat16), svh,
                                         preferred_element_type=jnp.float32)
    out_ref[0] = jnp.dot(acc_ref[...].astype(jnp.bfloat16), wo_ref[...],
                         preferred_element_type=jnp.float32)


def kernel(x, W_q, W_k, W_v, W_o, w_e, b_e):
    B, S, D = x.shape
    K = MAX_STATES
    D2 = 2 * D
    BS = 512 if S % 512 == 0 else S
    ns = S // BS
    base = ENERGY_BUDGET / S

    we_row = w_e.reshape(1, D)
    be_11 = b_e.reshape(1, 1)
    wq_h = W_q.astype(jnp.bfloat16)
    wo_h = W_o.astype(jnp.bfloat16)

    full = lambda b, s: (0, 0)
    q, kvw, idx, cntp = pl.pallas_call(
        functools.partial(_tc1_body, base=base),
        grid=(B, ns),
        in_specs=[
            pl.BlockSpec((1, BS, D), lambda b, s: (b, s, 0)),
            pl.BlockSpec((D, D), full),
            pl.BlockSpec((D, D), full),
            pl.BlockSpec((D, D), full),
            pl.BlockSpec((1, D), full),
            pl.BlockSpec((1, 1), full),
        ],
        out_specs=[
            pl.BlockSpec((1, BS, D), lambda b, s: (b, s, 0)),
            pl.BlockSpec((BS, D2), lambda b, s: (b * ns + s, 0)),
            pl.BlockSpec((1, 1, BS), lambda b, s: (b * ns + s, 0, 0)),
            pl.BlockSpec((1, 8, K), lambda b, s: (b, 0, 0)),
        ],
        out_shape=[
            jax.ShapeDtypeStruct((B, S, D), jnp.bfloat16),
            jax.ShapeDtypeStruct((B * S, D2), jnp.float32),
            jax.ShapeDtypeStruct((B * ns, 1, BS), jnp.int32),
            jax.ShapeDtypeStruct((B, 8, K), jnp.float32),
        ],
        scratch_shapes=[pltpu.VMEM((K, D), jnp.float32)],
        compiler_params=pltpu.CompilerParams(
            dimension_semantics=("arbitrary", "arbitrary")),
    )(x, wq_h, W_k, W_v, we_row, be_11)

    idx_flat = idx.reshape(B * S)
    parts = _make_sc_scatter(B * S, D2, B * K)(kvw, idx_flat)
    parts4 = parts.reshape(2, B, K, D2)

    out = pl.pallas_call(
        _tc2_body,
        grid=(B, ns),
        in_specs=[
            pl.BlockSpec((1, BS, D), lambda b, s: (b, s, 0)),
            pl.BlockSpec((2, 1, K, D2), lambda b, s: (0, b, 0, 0)),
            pl.BlockSpec((1, 8, K), lambda b, s: (b, 0, 0)),
            pl.BlockSpec((D, D), full),
        ],
        out_specs=pl.BlockSpec((1, BS, D), lambda b, s: (b, s, 0)),
        out_shape=jax.ShapeDtypeStruct((B, S, D), jnp.float32),
        scratch_shapes=[pltpu.VMEM((BS, D), jnp.float32)],
        compiler_params=pltpu.CompilerParams(
            dimension_semantics=("arbitrary", "arbitrary")),
    )(q, parts4, cntp, wo_h)
    return out


# speedup vs baseline: 2.6262x; 2.6262x over previous
import functools

import jax
import jax.numpy as jnp
from jax import lax
from jax.experimental import pallas as pl
from jax.experimental.pallas import tpu as pltpu

D_MODEL = 768
N_HEADS = 12
MAX_STATES = 64
ENERGY_BUDGET = 100.0
EPS = 1e-6


def _body(x_ref, wq_ref, wk_ref, wv_ref, wo_ref, we_ref, be_ref,
          out_ref, q_ref, sk_ref, sv_ref, cnt_ref, cn_ref, acc_ref,
          *, base, ns, bs):
    t = pl.program_id(1)
    K = MAX_STATES
    H = N_HEADS
    dh = D_MODEL // H

    @pl.when(t < ns)
    def _phase0():
        x = x_ref[0]
        xh = x.astype(jnp.bfloat16)
        q = jnp.dot(xh, wq_ref[...], preferred_element_type=jnp.float32)
        k = jnp.dot(x, wk_ref[...], preferred_element_type=jnp.float32)
        v = jnp.dot(xh, wv_ref[...], preferred_element_type=jnp.float32)
        q_ref[pl.ds(t * bs, bs), :] = q.astype(jnp.bfloat16)

        dot = jnp.sum(x * we_ref[...], axis=-1, keepdims=True) + be_ref[0, 0]
        w = base * 2.0 * jax.nn.sigmoid(dot)

        kn = k / (jnp.sqrt(jnp.sum(k * k, axis=-1, keepdims=True)) + EPS)

        @pl.when(t == 0)
        def _():
            cn_ref[...] = kn[:K]
            sk_ref[...] = jnp.zeros_like(sk_ref)
            sv_ref[...] = jnp.zeros_like(sv_ref)
            cnt_ref[...] = jnp.zeros_like(cnt_ref)

        sims = lax.dot_general(kn, cn_ref[...], (((1,), (1,)), ((), ())),
                               preferred_element_type=jnp.float32)
        m = jnp.max(sims, axis=-1, keepdims=True)
        j = lax.broadcasted_iota(jnp.int32, sims.shape, 1)
        assign = jnp.min(jnp.where(sims >= m, j, K), axis=-1, keepdims=True)
        p = (j == assign).astype(jnp.float32)

        sk_ref[...] += lax.dot_general(p, k * w, (((0,), (0,)), ((), ())),
                                       preferred_element_type=jnp.float32)
        sv_ref[...] += lax.dot_general(p, v * w, (((0,), (0,)), ((), ())),
                                       preferred_element_type=jnp.float32)
        cnt_ref[0:1, :] += jnp.sum(p * w, axis=0, keepdims=True)

    @pl.when(t >= ns)
    def _phase1():
        inv = 1.0 / (cnt_ref[0:1, :] + EPS)
        scale = inv / jnp.sqrt(jnp.float32(dh))
        q = q_ref[pl.ds((t - ns) * bs, bs), :]
        for h in range(H):
            lo = h * dh
            qh = q[:, lo:lo + dh]
            skh = sk_ref[:, lo:lo + dh].astype(jnp.bfloat16)
            svh = sv_ref[:, lo:lo + dh].astype(jnp.bfloat16)
            scores = lax.dot_general(qh, skh, (((1,), (1,)), ((), ())),
                                     preferred_element_type=jnp.float32) * scale
            mx = jnp.max(scores, axis=-1, keepdims=True)
            e = jnp.exp(scores - mx)
            attn = (e / jnp.sum(e, axis=-1, keepdims=True)) * inv
            acc_ref[:, lo:lo + dh] = jnp.dot(attn.astype(jnp.bfloat16), svh,
                                             preferred_element_type=jnp.float32)
        out_ref[0] = jnp.dot(acc_ref[...].astype(jnp.bfloat16), wo_ref[...],
                             preferred_element_type=jnp.float32)


def kernel(x, W_q, W_k, W_v, W_o, w_e, b_e):
    B, S, D = x.shape
    K = MAX_STATES
    BS = 1024 if S % 1024 == 0 else (512 if S % 512 == 0 else S)
    ns = S // BS
    base = ENERGY_BUDGET / S

    we_row = w_e.reshape(1, D)
    be_11 = b_e.reshape(1, 1)
    wq_h = W_q.astype(jnp.bfloat16)
    wv_h = W_v.astype(jnp.bfloat16)
    wo_h = W_o.astype(jnp.bfloat16)

    full = lambda b, t: (0, 0)
    out = pl.pallas_call(
        functools.partial(_body, base=base, ns=ns, bs=BS),
        grid=(B, 2 * ns),
        in_specs=[
            pl.BlockSpec((1, BS, D),
                         lambda b, t: (b, jnp.minimum(t, ns - 1), 0)),
            pl.BlockSpec((D, D), full),
            pl.BlockSpec((D, D), full),
            pl.BlockSpec((D, D), full),
            pl.BlockSpec((D, D), full),
            pl.BlockSpec((1, D), full),
            pl.BlockSpec((1, 1), full),
        ],
        out_specs=pl.BlockSpec((1, BS, D),
                               lambda b, t: (b, jnp.maximum(t - ns, 0), 0)),
        out_shape=jax.ShapeDtypeStruct((B, S, D), jnp.float32),
        scratch_shapes=[
            pltpu.VMEM((S, D), jnp.bfloat16),
            pltpu.VMEM((K, D), jnp.float32),
            pltpu.VMEM((K, D), jnp.float32),
            pltpu.VMEM((8, K), jnp.float32),
            pltpu.VMEM((K, D), jnp.float32),
            pltpu.VMEM((BS, D), jnp.float32),
        ],
        compiler_params=pltpu.CompilerParams(
            dimension_semantics=("arbitrary", "arbitrary")),
    )(x, wq_h, W_k, wv_h, wo_h, we_row, be_11)
    return out
